# baseline (speedup 1.0000x reference)
import functools

import numpy as np
import jax
import jax.numpy as jnp
from jax import lax
from jax.experimental import pallas as pl
from jax.experimental.pallas import tpu as pltpu
from jax.experimental.pallas import tpu_sc as plsc

N = 10000
E = 320000
D = 128
DE = 16
H = 5
C = 10
HID = 50
G = 64
DU = 32

NC = 2
NS = 16
NW = NC * NS
EPW = 10080
ETOT = NW * EPW
EPAD = ETOT - E
SUB = 48
NSUB = EPW // SUB
WROW = 80
GROW = 128
AROW = 128
NPAD = 10240
RPT = NPAD // NS

_j = np.arange(WROW)
_k = _j // 16
_g = (_j % 16) // 8
_h = _j % 8
_VALID = _h < H
_COLMAP = np.where(_VALID, _h * C + 2 * _k + _g, 0)

_P = np.zeros((WROW, HID), np.float32)
for _jj in range(WROW):
    if _VALID[_jj]:
        _P[_jj, _COLMAP[_jj]] = 1.0

_Q = np.zeros((16, WROW), np.float32)
for _jj in range(WROW):
    if _VALID[_jj]:
        _Q[_jj % 8, _jj] = 1.0


def _pad_weight(w, width):
    wp = w[:, _COLMAP] * _VALID[None, :].astype(w.dtype)
    if width > WROW:
        wp = jnp.pad(wp, ((0, 0), (0, width - WROW)))
    return wp


def _proj_body(x_ref, ws_ref, wd_ref, xl_ref, xr_ref):
    xv = x_ref[...]
    xl_ref[...] = jnp.dot(xv, ws_ref[...], preferred_element_type=jnp.float32)
    xr_ref[...] = jnp.dot(xv, wd_ref[...], preferred_element_type=jnp.float32)


def _ea_body(e_ref, w_ref, o_ref):
    o_ref[...] = jnp.dot(e_ref[...], w_ref[...],
                         preferred_element_type=jnp.float32)


def _mlp_body(pp_ref, x_ref, u_ref, b_ref, q_ref, w1a_ref, w1b_ref, w1c_ref,
              b1_ref, gamma_ref, beta_ref, w2_ref, b2_ref, out_ref):
    acc = pp_ref[0, :N] + pp_ref[1, :N]
    num = acc[:, :WROW]
    den16 = acc[:, WROW:WROW + 16]
    den80 = jnp.dot(den16, q_ref[...], preferred_element_type=jnp.float32)
    attn80 = num / (den80 + 1e-16)
    bb = b_ref[...]
    oh = (bb == lax.broadcasted_iota(jnp.int32, (1, G), 1)).astype(jnp.float32)
    ub = jnp.dot(oh, u_ref[...], preferred_element_type=jnp.float32)
    h1 = (jnp.dot(attn80, w1a_ref[...], preferred_element_type=jnp.float32)
          + jnp.dot(x_ref[...], w1b_ref[...], preferred_element_type=jnp.float32)
          + jnp.dot(ub, w1c_ref[...], preferred_element_type=jnp.float32)
          + b1_ref[...])
    scale = 1.0507009873554805
    alpha = 1.6732632423543772
    h1 = scale * jnp.where(h1 > 0, h1, alpha * (jnp.exp(h1) - 1.0))
    mean = jnp.mean(h1, axis=0, keepdims=True)
    var = jnp.mean((h1 - mean) ** 2, axis=0, keepdims=True)
    h1 = (h1 - mean) / jnp.sqrt(var + 1e-5) * gamma_ref[...] + beta_ref[...]
    out_ref[...] = jnp.dot(h1, w2_ref[...],
                           preferred_element_type=jnp.float32) + b2_ref[...]


def _make_edge_kernel():
    mesh = plsc.VectorSubcoreMesh(core_axis_name="c", subcore_axis_name="s")

    @functools.partial(
        pl.kernel,
        mesh=mesh,
        out_type=jax.ShapeDtypeStruct((NC, NPAD, AROW), jnp.float32),
        scratch_types=[
            pltpu.VMEM((SUB,), jnp.int32),
            pltpu.VMEM((SUB,), jnp.int32),
            pltpu.VMEM((SUB,), jnp.int32),
            pltpu.VMEM((SUB,), jnp.int32),
            pltpu.VMEM((SUB, GROW), jnp.float32),
            pltpu.VMEM((SUB, GROW), jnp.float32),
            pltpu.VMEM((SUB, GROW), jnp.float32),
            pltpu.VMEM((SUB, GROW), jnp.float32),
            pltpu.VMEM((SUB, WROW), jnp.float32),
            pltpu.VMEM((SUB, WROW), jnp.float32),
            pltpu.VMEM((SUB, AROW), jnp.float32),
            pltpu.VMEM((WROW,), jnp.float32),
            pltpu.VMEM_SHARED((NPAD, AROW), jnp.float32),
        ] + [pltpu.SemaphoreType.DMA] * 10,
    )
    def edge_kernel(xl_hbm, xr_hbm, ea_hbm, src_hbm, dst_hbm, att_hbm,
                    zeros_hbm, out_hbm, si0, si1, di0, di1,
                    el0_v, el1_v, er0_v, er1_v, ea0_v, ea1_v,
                    val_v, att_v, accum, *sems):
        c = lax.axis_index("c")
        s = lax.axis_index("s")
        wid = c * NS + s
        r0 = s * RPT
        pltpu.sync_copy(zeros_hbm.at[pl.ds(r0, RPT)], accum.at[pl.ds(r0, RPT)])
        pltpu.sync_copy(att_hbm, att_v)
        plsc.subcore_barrier()

        base0 = wid * EPW
        lane = lax.iota(jnp.int32, 16)
        rot8 = lax.bitwise_xor(lane, jnp.int32(8))
        denmask = lane < H
        dnums = lax.GatherDimensionNumbers(
            offset_dims=(), collapsed_slice_dims=(0,), start_index_map=(0,))
        atts = [att_v[pl.ds(16 * k, 16)] for k in range(5)]
        bufs = ((si0, di0, el0_v, er0_v, ea0_v, sems[0:5]),
                (si1, di1, el1_v, er1_v, ea1_v, sems[5:10]))

        def issue_idx(q, b):
            si, di, _, _, _, sm = bufs[b]
            base = base0 + q * SUB
            h1 = pltpu.async_copy(src_hbm.at[pl.ds(base, SUB)], si, sm[0])
            h2 = pltpu.async_copy(dst_hbm.at[pl.ds(base, SUB)], di, sm[1])
            return (h1, h2)

        def issue_gather(q, b):
            si, di, el_d, er_d, ea_d, sm = bufs[b]
            base = base0 + q * SUB
            h1 = pltpu.async_copy(xl_hbm.at[si], el_d, sm[2])
            h2 = pltpu.async_copy(xr_hbm.at[di], er_d, sm[3])
            h3 = pltpu.async_copy(ea_hbm.at[pl.ds(base, SUB)], ea_d, sm[4])
            return (h1, h2, h3)

        def ztail_body(i, carry):
            val_v[i, pl.ds(96, 16)] = jnp.zeros((16,), jnp.float32)
            val_v[i, pl.ds(112, 16)] = jnp.zeros((16,), jnp.float32)
            return carry

        lax.fori_loop(0, SUB, ztail_body, 0)

        def compute_scatter(b):
            _, di, el_d, er_d, ea_d, _ = bufs[b]

            @plsc.parallel_loop(0, SUB, 1, unroll=8)
            def _(i):
                t = jnp.zeros((16,), jnp.float32)
                els = []
                for k in range(5):
                    el_k = el_d[i, pl.ds(16 * k, 16)]
                    er_k = er_d[i, pl.ds(16 * k, 16)]
                    ea_k = ea_d[i, pl.ds(16 * k, 16)]
                    z = el_k + er_k + ea_k
                    m = jnp.maximum(z, 0.2 * z)
                    t = t + m * atts[k]
                    els.append(el_k)
                t_rot = lax.gather(t, rot8[:, None], dnums, slice_sizes=(1,),
                                   mode=lax.GatherScatterMode.PROMISE_IN_BOUNDS)
                ex = jnp.exp(t + t_rot)
                for k in range(5):
                    val_v[i, pl.ds(16 * k, 16)] = els[k] * ex
                val_v[i, pl.ds(WROW, 16)] = jnp.where(denmask, ex, 0.0)

            pltpu.sync_copy(val_v, accum.at[di], add=True)

        def pair_body(j, carry):
            q0 = 2 * j
            hi0 = issue_idx(q0, 0)
            hi1 = issue_idx(q0 + 1, 1)
            for h in hi0:
                h.wait()
            hg0 = issue_gather(q0, 0)
            for h in hi1:
                h.wait()
            hg1 = issue_gather(q0 + 1, 1)
            for h in hg0:
                h.wait()
            compute_scatter(0)
            for h in hg1:
                h.wait()
            compute_scatter(1)
            return carry

        lax.fori_loop(0, NSUB // 2, pair_body, 0)
        plsc.subcore_barrier()
        pltpu.sync_copy(accum.at[pl.ds(r0, RPT)], out_hbm.at[c, pl.ds(r0, RPT)])

    return edge_kernel


_edge_kernel = _make_edge_kernel()


def kernel(x, edge_index, edge_attr, u, batch, W_src, W_dst, W_edge, att,
           b_gat, W1, b1, gamma, beta, W2, b2):
    src = edge_index[0]
    dst = edge_index[1]

    ws_pad = _pad_weight(W_src, GROW)
    wd_pad = _pad_weight(W_dst, GROW)
    we_pad = _pad_weight(W_edge, WROW)
    att_pad = (att.reshape(-1)[_COLMAP] * _VALID.astype(jnp.float32))

    xl, xr = pl.pallas_call(
        _proj_body,
        out_shape=(jax.ShapeDtypeStruct((N, GROW), jnp.float32),
                   jax.ShapeDtypeStruct((N, GROW), jnp.float32)),
    )(x, ws_pad, wd_pad)
    xl = jnp.pad(xl, ((0, NPAD - N), (0, 0)))
    xr = jnp.pad(xr, ((0, NPAD - N), (0, 0)))

    edge_attr_p = jnp.pad(edge_attr, ((0, EPAD), (0, 0)))
    EB = 20160
    ea = pl.pallas_call(
        _ea_body,
        grid=(ETOT // EB,),
        in_specs=[pl.BlockSpec((EB, DE), lambda i: (i, 0)),
                  pl.BlockSpec((DE, WROW), lambda i: (0, 0))],
        out_specs=pl.BlockSpec((EB, WROW), lambda i: (i, 0)),
        out_shape=jax.ShapeDtypeStruct((ETOT, WROW), jnp.float32),
    )(edge_attr_p, we_pad)

    src_p = jnp.concatenate(
        [src, (jnp.arange(EPAD, dtype=jnp.int32) * 97) % N])
    dst_p = jnp.concatenate(
        [dst, N + (jnp.arange(EPAD, dtype=jnp.int32) % (NPAD - N))])
    zeros = jnp.zeros((NPAD, AROW), jnp.float32)
    partials = _edge_kernel(xl, xr, ea, src_p, dst_p, att_pad, zeros)

    w1a = jnp.asarray(_P) @ W1[:HID]
    w1b = W1[HID:HID + D]
    w1c = W1[HID + D:]
    b1p = (b1 + b_gat @ W1[:HID]).reshape(1, -1)

    out = pl.pallas_call(
        _mlp_body,
        out_shape=jax.ShapeDtypeStruct((N, D), jnp.float32),
    )(partials, x, u, batch.reshape(N, 1), jnp.asarray(_Q), w1a, w1b, w1c,
      b1p, gamma.reshape(1, -1), beta.reshape(1, -1), W2, b2.reshape(1, -1))
    return out


# speedup vs baseline: 1.0303x; 1.0303x over previous
import functools

import numpy as np
import jax
import jax.numpy as jnp
from jax import lax
from jax.experimental import pallas as pl
from jax.experimental.pallas import tpu as pltpu
from jax.experimental.pallas import tpu_sc as plsc

N = 10000
E = 320000
D = 128
DE = 16
H = 5
C = 10
HID = 50
G = 64
DU = 32

NC = 2
NS = 16
NW = NC * NS
EPW = 10080
ETOT = NW * EPW
EPAD = ETOT - E
SUB = 48
NSUB = EPW // SUB
WROW = 80
GROW = 128
AROW = 128
NPAD = 10240
RPT = NPAD // NS

_j = np.arange(WROW)
_k = _j // 16
_g = (_j % 16) // 8
_h = _j % 8
_VALID = _h < H
_COLMAP = np.where(_VALID, _h * C + 2 * _k + _g, 0)

_P = np.zeros((WROW, HID), np.float32)
for _jj in range(WROW):
    if _VALID[_jj]:
        _P[_jj, _COLMAP[_jj]] = 1.0

_Q = np.zeros((16, WROW), np.float32)
for _jj in range(WROW):
    if _VALID[_jj]:
        _Q[_jj % 8, _jj] = 1.0


def _pad_weight(w, width):
    wp = w[:, _COLMAP] * _VALID[None, :].astype(w.dtype)
    if width > WROW:
        wp = jnp.pad(wp, ((0, 0), (0, width - WROW)))
    return wp


def _proj_body(x_ref, ws_ref, wd_ref, xl_ref, xr_ref):
    xv = x_ref[...]
    xl_ref[...] = jnp.dot(xv, ws_ref[...], preferred_element_type=jnp.float32)
    xr_ref[...] = jnp.dot(xv, wd_ref[...], preferred_element_type=jnp.float32)


def _ea_body(e_ref, w_ref, o_ref):
    o_ref[...] = jnp.dot(e_ref[...], w_ref[...],
                         preferred_element_type=jnp.float32)


def _mlp_body(pp_ref, x_ref, u_ref, b_ref, q_ref, w1a_ref, w1b_ref, w1c_ref,
              b1_ref, gamma_ref, beta_ref, w2_ref, b2_ref, out_ref):
    acc = pp_ref[0, :N] + pp_ref[1, :N]
    num = acc[:, :WROW]
    den16 = acc[:, WROW:WROW + 16]
    den80 = jnp.dot(den16, q_ref[...], preferred_element_type=jnp.float32)
    attn80 = num / (den80 + 1e-16)
    bb = b_ref[...]
    oh = (bb == lax.broadcasted_iota(jnp.int32, (1, G), 1)).astype(jnp.float32)
    ub = jnp.dot(oh, u_ref[...], preferred_element_type=jnp.float32)
    h1 = (jnp.dot(attn80, w1a_ref[...], preferred_element_type=jnp.float32)
          + jnp.dot(x_ref[...], w1b_ref[...], preferred_element_type=jnp.float32)
          + jnp.dot(ub, w1c_ref[...], preferred_element_type=jnp.float32)
          + b1_ref[...])
    scale = 1.0507009873554805
    alpha = 1.6732632423543772
    h1 = scale * jnp.where(h1 > 0, h1, alpha * (jnp.exp(h1) - 1.0))
    mean = jnp.mean(h1, axis=0, keepdims=True)
    var = jnp.mean((h1 - mean) ** 2, axis=0, keepdims=True)
    h1 = (h1 - mean) / jnp.sqrt(var + 1e-5) * gamma_ref[...] + beta_ref[...]
    out_ref[...] = jnp.dot(h1, w2_ref[...],
                           preferred_element_type=jnp.float32) + b2_ref[...]


def _make_edge_kernel():
    mesh = plsc.VectorSubcoreMesh(core_axis_name="c", subcore_axis_name="s")

    @functools.partial(
        pl.kernel,
        mesh=mesh,
        out_type=jax.ShapeDtypeStruct((NC, NPAD, AROW), jnp.float32),
        scratch_types=[
            pltpu.VMEM((SUB,), jnp.int32),
            pltpu.VMEM((SUB,), jnp.int32),
            pltpu.VMEM((SUB,), jnp.int32),
            pltpu.VMEM((SUB,), jnp.int32),
            pltpu.VMEM((SUB, GROW), jnp.float32),
            pltpu.VMEM((SUB, GROW), jnp.float32),
            pltpu.VMEM((SUB, GROW), jnp.float32),
            pltpu.VMEM((SUB, GROW), jnp.float32),
            pltpu.VMEM((SUB, WROW), jnp.float32),
            pltpu.VMEM((SUB, WROW), jnp.float32),
            pltpu.VMEM((SUB, AROW), jnp.float32),
            pltpu.VMEM((WROW,), jnp.float32),
            pltpu.VMEM_SHARED((NPAD, AROW), jnp.float32),
        ] + [pltpu.SemaphoreType.DMA] * 10,
    )
    def edge_kernel(xl_hbm, xr_hbm, ea_hbm, src_hbm, dst_hbm, att_hbm,
                    zeros_hbm, out_hbm, si0, si1, di0, di1,
                    el0_v, el1_v, er0_v, er1_v, ea0_v, ea1_v,
                    val_v, att_v, accum, *sems):
        c = lax.axis_index("c")
        s = lax.axis_index("s")
        wid = c * NS + s
        r0 = s * RPT
        pltpu.sync_copy(zeros_hbm.at[pl.ds(r0, RPT)], accum.at[pl.ds(r0, RPT)])
        pltpu.sync_copy(att_hbm, att_v)
        plsc.subcore_barrier()

        base0 = wid * EPW
        lane = lax.iota(jnp.int32, 16)
        rot8 = lax.bitwise_xor(lane, jnp.int32(8))
        denmask = lane < H
        dnums = lax.GatherDimensionNumbers(
            offset_dims=(), collapsed_slice_dims=(0,), start_index_map=(0,))
        atts = [att_v[pl.ds(16 * k, 16)] for k in range(5)]
        bufs = ((si0, di0, el0_v, er0_v, ea0_v, sems[0:5]),
                (si1, di1, el1_v, er1_v, ea1_v, sems[5:10]))

        def issue_idx(q, b):
            si, di, _, _, _, sm = bufs[b]
            base = base0 + q * SUB
            h1 = pltpu.async_copy(src_hbm.at[pl.ds(base, SUB)], si, sm[0])
            h2 = pltpu.async_copy(dst_hbm.at[pl.ds(base, SUB)], di, sm[1])
            return (h1, h2)

        def issue_gather(q, b):
            si, di, el_d, er_d, ea_d, sm = bufs[b]
            base = base0 + q * SUB
            h1 = pltpu.async_copy(xl_hbm.at[si], el_d, sm[2])
            h2 = pltpu.async_copy(xr_hbm.at[di], er_d, sm[3])
            h3 = pltpu.async_copy(ea_hbm.at[pl.ds(base, SUB)], ea_d, sm[4])
            return (h1, h2, h3)

        def ztail_body(i, carry):
            val_v[i, pl.ds(96, 16)] = jnp.zeros((16,), jnp.float32)
            val_v[i, pl.ds(112, 16)] = jnp.zeros((16,), jnp.float32)
            return carry

        lax.fori_loop(0, SUB, ztail_body, 0)

        def compute_scatter(b):
            _, di, el_d, er_d, ea_d, _ = bufs[b]

            @plsc.parallel_loop(0, SUB, 1, unroll=4)
            def _(i):
                t = jnp.zeros((16,), jnp.float32)
                els = []
                for k in range(5):
                    el_k = el_d[i, pl.ds(16 * k, 16)]
                    er_k = er_d[i, pl.ds(16 * k, 16)]
                    ea_k = ea_d[i, pl.ds(16 * k, 16)]
                    z = el_k + er_k + ea_k
                    m = jnp.maximum(z, 0.2 * z)
                    t = t + m * atts[k]
                    els.append(el_k)
                t_rot = lax.gather(t, rot8[:, None], dnums, slice_sizes=(1,),
                                   mode=lax.GatherScatterMode.PROMISE_IN_BOUNDS)
                ex = jnp.exp(t + t_rot)
                for k in range(5):
                    val_v[i, pl.ds(16 * k, 16)] = els[k] * ex
                val_v[i, pl.ds(WROW, 16)] = jnp.where(denmask, ex, 0.0)

            pltpu.sync_copy(val_v, accum.at[di], add=True)

        def pair_body(j, carry):
            q0 = 2 * j
            hi0 = issue_idx(q0, 0)
            hi1 = issue_idx(q0 + 1, 1)
            for h in hi0:
                h.wait()
            hg0 = issue_gather(q0, 0)
            for h in hi1:
                h.wait()
            hg1 = issue_gather(q0 + 1, 1)
            for h in hg0:
                h.wait()
            compute_scatter(0)
            for h in hg1:
                h.wait()
            compute_scatter(1)
            return carry

        lax.fori_loop(0, NSUB // 2, pair_body, 0)
        plsc.subcore_barrier()
        pltpu.sync_copy(accum.at[pl.ds(r0, RPT)], out_hbm.at[c, pl.ds(r0, RPT)])

    return edge_kernel


_edge_kernel = _make_edge_kernel()


def kernel(x, edge_index, edge_attr, u, batch, W_src, W_dst, W_edge, att,
           b_gat, W1, b1, gamma, beta, W2, b2):
    src = edge_index[0]
    dst = edge_index[1]

    ws_pad = _pad_weight(W_src, GROW)
    wd_pad = _pad_weight(W_dst, GROW)
    we_pad = _pad_weight(W_edge, WROW)
    att_pad = (att.reshape(-1)[_COLMAP] * _VALID.astype(jnp.float32))

    xl, xr = pl.pallas_call(
        _proj_body,
        out_shape=(jax.ShapeDtypeStruct((N, GROW), jnp.float32),
                   jax.ShapeDtypeStruct((N, GROW), jnp.float32)),
    )(x, ws_pad, wd_pad)
    xl = jnp.pad(xl, ((0, NPAD - N), (0, 0)))
    xr = jnp.pad(xr, ((0, NPAD - N), (0, 0)))

    edge_attr_p = jnp.pad(edge_attr, ((0, EPAD), (0, 0)))
    EB = 20160
    ea = pl.pallas_call(
        _ea_body,
        grid=(ETOT // EB,),
        in_specs=[pl.BlockSpec((EB, DE), lambda i: (i, 0)),
                  pl.BlockSpec((DE, WROW), lambda i: (0, 0))],
        out_specs=pl.BlockSpec((EB, WROW), lambda i: (i, 0)),
        out_shape=jax.ShapeDtypeStruct((ETOT, WROW), jnp.float32),
    )(edge_attr_p, we_pad)

    src_p = jnp.concatenate(
        [src, (jnp.arange(EPAD, dtype=jnp.int32) * 97) % N])
    dst_p = jnp.concatenate(
        [dst, N + (jnp.arange(EPAD, dtype=jnp.int32) % (NPAD - N))])
    zeros = jnp.zeros((NPAD, AROW), jnp.float32)
    partials = _edge_kernel(xl, xr, ea, src_p, dst_p, att_pad, zeros)

    w1a = jnp.asarray(_P) @ W1[:HID]
    w1b = W1[HID:HID + D]
    w1c = W1[HID + D:]
    b1p = (b1 + b_gat @ W1[:HID]).reshape(1, -1)

    out = pl.pallas_call(
        _mlp_body,
        out_shape=jax.ShapeDtypeStruct((N, D), jnp.float32),
    )(partials, x, u, batch.reshape(N, 1), jnp.asarray(_Q), w1a, w1b, w1c,
      b1p, gamma.reshape(1, -1), beta.reshape(1, -1), W2, b2.reshape(1, -1))
    return out
